# Initial kernel scaffold; baseline (speedup 1.0000x reference)
import jax
import jax.numpy as jnp
from jax import lax
from jax.experimental import pallas as pl
from jax.experimental.pallas import tpu as pltpu
from jax.experimental.pallas import tpu_sc as plsc

B = 16384
N_NUM = 13
NUM_OUT = 64
N_CAT = 26
VOCAB = 1000
EMB = 32
OUT_D = NUM_OUT + N_CAT * EMB

NC = 2
NS = 16
NW = NC * NS
ROWS_PER_W = B // NW
CHUNK = 128
N_CHUNKS = ROWS_PER_W // CHUNK
LANES = 16


def _mm_body(x_ref, w_ref, b_ref, o_ref):
    o_ref[...] = (
        jnp.dot(x_ref[...], w_ref[...], preferred_element_type=jnp.float32)
        + b_ref[...]
    )


def _num_matmul(x, W, b2):
    MB = 2048
    return pl.pallas_call(
        _mm_body,
        grid=(B // MB,),
        in_specs=[
            pl.BlockSpec((MB, N_NUM), lambda i: (i, 0)),
            pl.BlockSpec((N_NUM, NUM_OUT), lambda i: (0, 0)),
            pl.BlockSpec((1, NUM_OUT), lambda i: (0, 0)),
        ],
        out_specs=pl.BlockSpec((MB, NUM_OUT), lambda i: (i, 0)),
        out_shape=jax.ShapeDtypeStruct((B, NUM_OUT), jnp.float32),
    )(x, W, b2)


def _sc_body(num_emb_hbm, catT_hbm, tables_hbm, out_hbm,
             idx_v, dest_v, num_v, gsem, osem, nsem):
    cid = lax.axis_index("c")
    sid = lax.axis_index("s")
    wid = sid * NC + cid
    row0 = wid * ROWS_PER_W

    def chunk_body(ci, carry):
        base = pl.multiple_of(row0 + ci * CHUNK, CHUNK)
        pltpu.sync_copy(catT_hbm.at[:, pl.ds(base, CHUNK)], idx_v)
        for f in range(N_CAT):
            off = f * VOCAB
            for j in range(CHUNK // LANES):
                sl = pl.ds(j * LANES, LANES)
                idx_v[f, sl] = idx_v[f, sl] + off
        gathers = [
            pltpu.async_copy(tables_hbm.at[idx_v.at[f]], dest_v.at[f], gsem)
            for f in range(N_CAT)
        ]
        pltpu.async_copy(num_emb_hbm.at[pl.ds(base, CHUNK)], num_v, nsem).wait()
        out_num = pltpu.async_copy(
            num_v, out_hbm.at[pl.ds(base, CHUNK), pl.ds(0, NUM_OUT)], nsem
        )
        outs = []
        for f in range(N_CAT):
            gathers[f].wait()
            outs.append(
                pltpu.async_copy(
                    dest_v.at[f],
                    out_hbm.at[
                        pl.ds(base, CHUNK), pl.ds(NUM_OUT + f * EMB, EMB)
                    ],
                    osem,
                )
            )
        out_num.wait()
        for o in outs:
            o.wait()
        return carry

    lax.fori_loop(0, N_CHUNKS, chunk_body, 0)


_sc_gather = pl.kernel(
    _sc_body,
    mesh=plsc.VectorSubcoreMesh(core_axis_name="c", subcore_axis_name="s"),
    out_type=jax.ShapeDtypeStruct((B, OUT_D), jnp.float32),
    scratch_types=[
        pltpu.VMEM((N_CAT, CHUNK), jnp.int32),
        pltpu.VMEM((N_CAT, CHUNK, EMB), jnp.float32),
        pltpu.VMEM((CHUNK, NUM_OUT), jnp.float32),
        pltpu.SemaphoreType.DMA,
        pltpu.SemaphoreType.DMA,
        pltpu.SemaphoreType.DMA,
    ],
)


@jax.jit
def kernel(num_tensor, cat_tensor, W, b, tables):
    num_emb = _num_matmul(num_tensor, W, b.reshape(1, NUM_OUT))
    catT = cat_tensor.T
    tables_flat = tables.reshape(N_CAT * VOCAB, EMB)
    return _sc_gather(num_emb, catT, tables_flat)


# speedup vs baseline: 10.1226x; 10.1226x over previous
import jax
import jax.numpy as jnp
from jax import lax
from jax.experimental import pallas as pl
from jax.experimental.pallas import tpu as pltpu
from jax.experimental.pallas import tpu_sc as plsc

B = 16384
N_NUM = 13
NUM_OUT = 64
N_CAT = 26
VOCAB = 1000
EMB = 32
OUT_D = NUM_OUT + N_CAT * EMB

NC = 2
NS = 16
NW = NC * NS
ROWS_PER_W = B // NW
CHUNK = 128
N_CHUNKS = ROWS_PER_W // CHUNK
LANES = 16


def _mm_body(x_ref, w_ref, b_ref, o_ref):
    o_ref[...] = (
        jnp.dot(x_ref[...], w_ref[...], preferred_element_type=jnp.float32)
        + b_ref[...]
    )


def _num_matmul(x, W, b2):
    MB = 2048
    return pl.pallas_call(
        _mm_body,
        grid=(B // MB,),
        in_specs=[
            pl.BlockSpec((MB, N_NUM), lambda i: (i, 0)),
            pl.BlockSpec((N_NUM, NUM_OUT), lambda i: (0, 0)),
            pl.BlockSpec((1, NUM_OUT), lambda i: (0, 0)),
        ],
        out_specs=pl.BlockSpec((MB, NUM_OUT), lambda i: (i, 0)),
        out_shape=jax.ShapeDtypeStruct((B, NUM_OUT), jnp.float32),
    )(x, W, b2)


def _sc_body(num_emb_hbm, catT_hbm, tables_hbm, out_hbm,
             idx_v, dest_v, num_v, gsem, osem, nsem):
    cid = lax.axis_index("c")
    sid = lax.axis_index("s")
    wid = sid * NC + cid
    row0 = wid * ROWS_PER_W

    def chunk_body(ci, carry):
        base = pl.multiple_of(row0 + ci * CHUNK, CHUNK)
        pltpu.sync_copy(catT_hbm.at[:, pl.ds(base, CHUNK)], idx_v)
        for f in range(N_CAT):
            off = f * VOCAB
            for j in range(CHUNK // LANES):
                sl = pl.ds(j * LANES, LANES)
                idx_v[f, sl] = idx_v[f, sl] + off
        gathers = [
            pltpu.async_copy(tables_hbm.at[idx_v.at[f]], dest_v.at[f], gsem)
            for f in range(N_CAT)
        ]
        pltpu.async_copy(num_emb_hbm.at[pl.ds(base, CHUNK)], num_v, nsem).wait()
        out_num = pltpu.async_copy(
            num_v, out_hbm.at[pl.ds(base, CHUNK), pl.ds(0, NUM_OUT)], nsem
        )
        outs = []
        for f in range(N_CAT):
            gathers[f].wait()
            outs.append(
                pltpu.async_copy(
                    dest_v.at[f],
                    out_hbm.at[
                        pl.ds(base, CHUNK), pl.ds(NUM_OUT + f * EMB, EMB)
                    ],
                    osem,
                )
            )
        out_num.wait()
        for o in outs:
            o.wait()
        return carry

    lax.fori_loop(0, N_CHUNKS, chunk_body, 0)


_sc_gather = pl.kernel(
    _sc_body,
    mesh=plsc.VectorSubcoreMesh(core_axis_name="c", subcore_axis_name="s"),
    compiler_params=pltpu.CompilerParams(use_tc_tiling_on_sc=False),
    out_type=jax.ShapeDtypeStruct((B, OUT_D), jnp.float32),
    scratch_types=[
        pltpu.VMEM((N_CAT, CHUNK), jnp.int32),
        pltpu.VMEM((N_CAT, CHUNK, EMB), jnp.float32),
        pltpu.VMEM((CHUNK, NUM_OUT), jnp.float32),
        pltpu.SemaphoreType.DMA,
        pltpu.SemaphoreType.DMA,
        pltpu.SemaphoreType.DMA,
    ],
)


@jax.jit
def kernel(num_tensor, cat_tensor, W, b, tables):
    num_emb = _num_matmul(num_tensor, W, b.reshape(1, NUM_OUT))
    catT = cat_tensor.T
    tables_flat = tables.reshape(N_CAT * VOCAB, EMB)
    return _sc_gather(num_emb, catT, tables_flat)
